# baseline (speedup 1.0000x reference)
import functools

import jax
import jax.numpy as jnp
from jax import lax
from jax.experimental import pallas as pl
from jax.experimental.pallas import tpu as pltpu
from jax.experimental.pallas import tpu_sc as plsc

NUM_NODES = 100000
VOCAB = 119
EMB_DIM = 128

NC = 2
NS = 16
NW = NC * NS

CB = 3200
SUB = 200
NSUB = CB // SUB
NBUF = 2

ROUTED2 = (3, 7, 11, 15)


def _routed(j):
    return j in ROUTED2


def _emb_body(idx_hbm, table_hbm, out1_hbm, out2_hbm, idx_v, table_sh, rows_v,
              spm, gsems, s1sems, d2sems, csems, s2sems):
    sid = lax.axis_index("s")
    wid = sid * NC + lax.axis_index("c")
    base = pl.multiple_of(jnp.minimum(wid * CB, NUM_NODES - CB), 8)

    @pl.when(sid == 0)
    def _():
        pltpu.sync_copy(table_hbm, table_sh)

    pltpu.sync_copy(idx_hbm.at[pl.ds(base, CB)], idx_v)
    plsc.subcore_barrier()

    def gather(j, b):
        return pltpu.make_async_copy(
            table_sh.at[idx_v.at[pl.ds(j * SUB, SUB)]], rows_v.at[b], gsems.at[b]
        )

    def direct1(j, b):
        return pltpu.make_async_copy(
            rows_v.at[b], out1_hbm.at[pl.ds(base + j * SUB, SUB)], s1sems.at[b]
        )

    def direct2(j, b):
        return pltpu.make_async_copy(
            rows_v.at[b], out2_hbm.at[pl.ds(base + j * SUB, SUB)], d2sems.at[b]
        )

    def stage(j, b):
        return pltpu.make_async_copy(rows_v.at[b], spm.at[sid], csems.at[b])

    def dma2(j, b):
        return pltpu.make_async_copy(
            spm.at[sid], out2_hbm.at[pl.ds(base + j * SUB, SUB)], s2sems.at[b]
        )

    gather(0, 0).start()
    pending_dma2 = None
    for j in range(NSUB):
        b = j % NBUF
        gather(j, b).wait()
        direct1(j, b).start()
        if _routed(j):
            if pending_dma2 is not None:
                dma2(pending_dma2, pending_dma2 % NBUF).wait()
            stage(j, b).start()
            stage(j, b).wait()
            dma2(j, b).start()
            pending_dma2 = j
        else:
            direct2(j, b).start()
        if j + 1 < NSUB:
            nb = (j + 1) % NBUF
            if j + 1 >= NBUF:
                jj = j + 1 - NBUF
                direct1(jj, nb).wait()
                if not _routed(jj):
                    direct2(jj, nb).wait()
            gather(j + 1, nb).start()
    for j in range(max(0, NSUB - NBUF), NSUB):
        b = j % NBUF
        direct1(j, b).wait()
        if not _routed(j):
            direct2(j, b).wait()
    if pending_dma2 is not None:
        dma2(pending_dma2, pending_dma2 % NBUF).wait()


def _emb_lookup(atomic_numbers, emb_table):
    mesh = plsc.VectorSubcoreMesh(core_axis_name="c", subcore_axis_name="s")
    fn = functools.partial(
        pl.kernel,
        mesh=mesh,
        out_type=(
            jax.ShapeDtypeStruct((NUM_NODES, EMB_DIM), jnp.float32),
            jax.ShapeDtypeStruct((NUM_NODES, EMB_DIM), jnp.float32),
        ),
        scratch_types=[
            pltpu.VMEM((CB,), jnp.int32),
            pltpu.VMEM_SHARED((VOCAB, EMB_DIM), jnp.float32),
            pltpu.VMEM((NBUF, SUB, EMB_DIM), jnp.float32),
            pltpu.VMEM_SHARED((NS, SUB, EMB_DIM), jnp.float32),
            pltpu.SemaphoreType.DMA((NBUF,)),
            pltpu.SemaphoreType.DMA((NBUF,)),
            pltpu.SemaphoreType.DMA((NBUF,)),
            pltpu.SemaphoreType.DMA((NBUF,)),
            pltpu.SemaphoreType.DMA((NBUF,)),
        ],
    )(_emb_body)
    return fn(atomic_numbers, emb_table)


def kernel(atomic_numbers, emb_table):
    out1, out2 = _emb_lookup(atomic_numbers.astype(jnp.int32), emb_table)
    return (out1, out2)


# speedup vs baseline: 1.0373x; 1.0373x over previous
import functools

import jax
import jax.numpy as jnp
from jax import lax
from jax.experimental import pallas as pl
from jax.experimental.pallas import tpu as pltpu
from jax.experimental.pallas import tpu_sc as plsc

NUM_NODES = 100000
VOCAB = 119
EMB_DIM = 128

NC = 2
NS = 16
NW = NC * NS

CB = 3200
SUB = 320
NSUB = CB // SUB
NBUF = 3


def _emb_body(idx_hbm, table_hbm, out1_hbm, out2_hbm, idx_v, rows_v, table_sh,
              gsems, ssems):
    sid = lax.axis_index("s")
    wid = sid * NC + lax.axis_index("c")
    base = pl.multiple_of(jnp.minimum(wid * CB, NUM_NODES - CB), 8)

    @pl.when(sid == 0)
    def _():
        pltpu.sync_copy(table_hbm, table_sh)

    pltpu.sync_copy(idx_hbm.at[pl.ds(base, CB)], idx_v)
    plsc.subcore_barrier()

    def gather(j, b):
        return pltpu.make_async_copy(
            table_sh.at[idx_v.at[pl.ds(j * SUB, SUB)]], rows_v.at[b], gsems.at[b]
        )

    def scatters(j, b):
        return [
            pltpu.make_async_copy(
                rows_v.at[b], out.at[pl.ds(base + j * SUB, SUB)], ssems.at[b]
            )
            for out in (out1_hbm, out2_hbm)
        ]

    gather(0, 0).start()
    for j in range(NSUB):
        b = j % NBUF
        gather(j, b).wait()
        if j + 1 < NSUB:
            nb = (j + 1) % NBUF
            if j + 1 >= NBUF:
                for cp in scatters(j + 1 - NBUF, nb):
                    cp.wait()
            gather(j + 1, nb).start()
        for cp in scatters(j, b):
            cp.start()
    for j in range(max(0, NSUB - NBUF), NSUB):
        for cp in scatters(j, j % NBUF):
            cp.wait()


@functools.partial(jax.jit, static_argnums=())
def _emb_lookup(atomic_numbers, emb_table):
    mesh = plsc.VectorSubcoreMesh(core_axis_name="c", subcore_axis_name="s")
    fn = functools.partial(
        pl.kernel,
        mesh=mesh,
        out_type=(
            jax.ShapeDtypeStruct((NUM_NODES, EMB_DIM), jnp.float32),
            jax.ShapeDtypeStruct((NUM_NODES, EMB_DIM), jnp.float32),
        ),
        scratch_types=[
            pltpu.VMEM((CB,), jnp.int32),
            pltpu.VMEM((NBUF, SUB, EMB_DIM), jnp.float32),
            pltpu.VMEM_SHARED((VOCAB, EMB_DIM), jnp.float32),
            pltpu.SemaphoreType.DMA((NBUF,)),
            pltpu.SemaphoreType.DMA((NBUF,)),
        ],
    )(_emb_body)
    return fn(atomic_numbers, emb_table)


def kernel(atomic_numbers, emb_table):
    out1, out2 = _emb_lookup(atomic_numbers.astype(jnp.int32), emb_table)
    return (out1, out2)
